# baseline (speedup 1.0000x reference)
import functools

import jax
import jax.numpy as jnp
from jax import lax
from jax.experimental import pallas as pl
from jax.experimental.pallas import tpu as pltpu
from jax.experimental.pallas import tpu_sc as plsc

_F = 26
_V = 100000
_D = 32
_B = 4096


@functools.cache
def _build():
    info = plsc.get_sparse_core_info()
    nl = info.num_lanes
    nw = info.num_cores * info.num_subcores
    bw = _B // nw

    mesh = plsc.VectorSubcoreMesh(core_axis_name="c", subcore_axis_name="s")

    @functools.partial(
        pl.kernel,
        mesh=mesh,
        compiler_params=pltpu.CompilerParams(
            use_tc_tiling_on_sc=False, needs_layout_passes=False
        ),
        out_type=jax.ShapeDtypeStruct((_F * _D, _B), jnp.float32),
        scratch_types=[
            pltpu.VMEM((_F, bw), jnp.int32),
            pltpu.VMEM((2, bw, _D), jnp.float32),
            pltpu.VMEM((2, _D, bw), jnp.float32),
            pltpu.SemaphoreType.DMA,
            pltpu.SemaphoreType.DMA,
        ],
    )
    def sc_gather(xt_hbm, tab_hbm, out_hbm, idx_v, gbuf, tbuf, gsem, wsem):
        w = lax.axis_index("s") * info.num_cores + lax.axis_index("c")
        b0 = w * bw
        pltpu.sync_copy(xt_hbm.at[:, pl.ds(b0, bw)], idx_v)

        def fire(f, slot):
            pltpu.async_copy(
                tab_hbm.at[f].at[idx_v.at[f]], gbuf.at[slot], gsem
            )

        fire(0, 0)

        def body(f, carry):
            slot = lax.rem(f, 2)
            pltpu.make_async_copy(
                tab_hbm.at[0].at[idx_v.at[0]], gbuf.at[slot], gsem
            ).wait()

            @pl.when(f + 1 < _F)
            def _():
                fire(f + 1, 1 - slot)

            @pl.when(f >= 2)
            def _():
                pltpu.make_async_copy(
                    tbuf.at[slot], out_hbm.at[pl.ds(0, _D), pl.ds(b0, bw)],
                    wsem,
                ).wait()

            def trans(d, c2):
                dvec = jnp.full((nl,), d, dtype=jnp.int32)
                for j in range(bw // nl):
                    bvec = lax.iota(jnp.int32, nl) + j * nl
                    vals = plsc.load_gather(gbuf.at[slot], [bvec, dvec])
                    tbuf[slot, d, pl.ds(j * nl, nl)] = vals
                return c2

            lax.fori_loop(0, _D, trans, 0)
            pltpu.async_copy(
                tbuf.at[slot],
                out_hbm.at[pl.ds(f * _D, _D), pl.ds(b0, bw)],
                wsem,
            )
            return carry

        lax.fori_loop(0, _F, body, 0)
        for _ in range(2):
            pltpu.make_async_copy(
                tbuf.at[0], out_hbm.at[pl.ds(0, _D), pl.ds(b0, bw)], wsem
            ).wait()

    return sc_gather


def kernel(x, tables):
    out_t = _build()(x.T, tables)
    return out_t.T.reshape(_B, _F * _D)


# speedup vs baseline: 1.0054x; 1.0054x over previous
import functools

import jax
import jax.numpy as jnp
from jax import lax
from jax.experimental import pallas as pl
from jax.experimental.pallas import tpu as pltpu
from jax.experimental.pallas import tpu_sc as plsc

_F = 26
_V = 100000
_D = 32
_B = 4096
_NL = 16


@functools.cache
def _build():
    info = plsc.get_sparse_core_info()
    nw = info.num_cores * info.num_subcores
    bw = _B // nw
    nv = bw // _NL

    mesh = plsc.VectorSubcoreMesh(core_axis_name="c", subcore_axis_name="s")

    @functools.partial(
        pl.kernel,
        mesh=mesh,
        compiler_params=pltpu.CompilerParams(
            use_tc_tiling_on_sc=True, needs_layout_passes=False
        ),
        out_type=jax.ShapeDtypeStruct((_F * _D, _B), jnp.float32),
        scratch_types=[
            pltpu.VMEM((_F, bw), jnp.int32),
            pltpu.VMEM((2, bw), jnp.int32),
            pltpu.VMEM((2, bw, 128), jnp.float32),
            pltpu.VMEM((2, _D, bw), jnp.float32),
            pltpu.SemaphoreType.DMA,
            pltpu.SemaphoreType.DMA,
        ],
    )
    def sc_gather(xt_hbm, tab_hbm, out_hbm, idx_v, ridx, gbuf, obuf, gsem, wsem):
        w = lax.axis_index("s") * info.num_cores + lax.axis_index("c")
        b0 = w * bw
        for f2 in range(_F // 2):
            pltpu.sync_copy(
                xt_hbm.at[pl.ds(f2 * 2, 2), pl.ds(b0, bw)],
                idx_v.at[pl.ds(f2 * 2, 2)],
            )

        def fire(f, slot):
            def mk(k, c):
                v = idx_v[f, pl.ds(k * _NL, _NL)]
                ridx[slot, pl.ds(k * _NL, _NL)] = (
                    lax.shift_right_logical(v, 2) + f * (_V // 4)
                )
                return c

            lax.fori_loop(0, nv, mk, 0)
            pltpu.async_copy(
                tab_hbm.at[ridx.at[slot]], gbuf.at[slot], gsem
            )

        fire(0, 0)

        def body(f, carry):
            slot = lax.rem(f, 2)
            pltpu.make_async_copy(
                tab_hbm.at[ridx.at[0]], gbuf.at[0], gsem
            ).wait()

            @pl.when(f + 1 < _F)
            def _():
                fire(f + 1, 1 - slot)

            @pl.when(f >= 2)
            def _():
                pltpu.make_async_copy(
                    obuf.at[0], out_hbm.at[pl.ds(0, _D), pl.ds(b0, bw)], wsem
                ).wait()

            def ext(k, c):
                bvec = lax.iota(jnp.int32, _NL) + k * _NL
                rem = lax.rem(idx_v[f, pl.ds(k * _NL, _NL)], 4) * _D

                def ext_d(d, c2):
                    vals = plsc.load_gather(gbuf.at[slot], [bvec, rem + d])
                    obuf[slot, d, pl.ds(k * _NL, _NL)] = vals
                    return c2

                lax.fori_loop(0, _D, ext_d, 0)
                return c

            lax.fori_loop(0, nv, ext, 0)
            pltpu.async_copy(
                obuf.at[slot],
                out_hbm.at[pl.ds(f * _D, _D), pl.ds(b0, bw)],
                wsem,
            )
            return carry

        lax.fori_loop(0, _F, body, 0)
        for _ in range(2):
            pltpu.make_async_copy(
                obuf.at[0], out_hbm.at[pl.ds(0, _D), pl.ds(b0, bw)], wsem
            ).wait()

    return sc_gather


def kernel(x, tables):
    out_t = _build()(x.T, tables.reshape(_F * _V // 4, 128))
    return out_t.T.reshape(_B, _F * _D)


# speedup vs baseline: 1.0319x; 1.0264x over previous
import functools

import jax
import jax.numpy as jnp
from jax import lax
from jax.experimental import pallas as pl
from jax.experimental.pallas import tpu as pltpu
from jax.experimental.pallas import tpu_sc as plsc

_F = 26
_V = 100000
_D = 32
_B = 4096
_CHUNK = 128


@functools.cache
def _build():
    info = plsc.get_sparse_core_info()
    nc, ns, nl = info.num_cores, info.num_subcores, info.num_lanes
    nw = nc * ns
    total = _B * _F
    per_w = total // nw
    assert per_w * nw == total and per_w % _F == 0 and per_w % _CHUNK == 0
    n_vec = per_w // nl
    n_gather = per_w // _CHUNK

    mesh = plsc.VectorSubcoreMesh(core_axis_name="c", subcore_axis_name="s")

    @functools.partial(
        pl.kernel,
        mesh=mesh,
        compiler_params=pltpu.CompilerParams(use_tc_tiling_on_sc=False),
        out_type=jax.ShapeDtypeStruct((total, _D), jnp.float32),
        scratch_types=[
            pltpu.VMEM((per_w,), jnp.int32),
            pltpu.VMEM((per_w, _D), jnp.float32),
            pltpu.SemaphoreType.DMA,
        ],
    )
    def sc_gather(x_hbm, tab_hbm, out_hbm, idx_v, rows_v, sem):
        wid = lax.axis_index("s") * nc + lax.axis_index("c")
        base = wid * per_w
        pltpu.sync_copy(x_hbm.at[pl.ds(base, per_w)], idx_v)

        def add_off(i, carry):
            pos = lax.iota(jnp.int32, nl) + i * nl
            off = lax.rem(pos, _F) * _V
            idx_v[pl.ds(i * nl, nl)] = idx_v[pl.ds(i * nl, nl)] + off
            return carry

        lax.fori_loop(0, n_vec, add_off, 0)

        def gather(j, carry):
            pltpu.async_copy(
                tab_hbm.at[idx_v.at[pl.ds(j * _CHUNK, _CHUNK)]],
                rows_v.at[pl.ds(j * _CHUNK, _CHUNK)],
                sem,
            ).wait()
            return carry

        lax.fori_loop(0, n_gather, gather, 0)
        pltpu.sync_copy(rows_v, out_hbm.at[pl.ds(base, per_w)])

    return sc_gather


def kernel(x, tables):
    out = _build()(x.reshape(-1), tables.reshape(_F * _V, _D))
    return out.reshape(_B, _F * _D)
